# baseline (speedup 1.0000x reference)
import functools

import jax
import jax.numpy as jnp
from jax.experimental import pallas as pl
from jax.experimental.pallas import tpu as pltpu
from jax.experimental.pallas import tpu_sc as plsc

B = 4096
S = 200
D = 128
GATHER_W = 128
BB = 32
NCH = 8
CB = B // NCH


def _sc_gather(tok_table, x_flat, n_rows):
    mesh = plsc.VectorSubcoreMesh(core_axis_name="c", subcore_axis_name="s")
    num_windows = n_rows // GATHER_W
    dw = tok_table.shape[1]

    @functools.partial(
        pl.kernel,
        out_type=jax.ShapeDtypeStruct((n_rows, dw), jnp.int32),
        mesh=mesh,
        compiler_params=pltpu.CompilerParams(use_tc_tiling_on_sc=False),
    )
    def gather_kernel(tok_hbm, idx_hbm, out_hbm):
        def body(idx_vmem, out_vmem):
            pltpu.sync_copy(tok_hbm.at[idx_vmem.at[0]], out_vmem)

        pltpu.emit_pipeline(
            body,
            grid=(num_windows,),
            in_specs=[pl.BlockSpec((1, GATHER_W), index_map=lambda i: (0, i))],
            out_specs=[pl.BlockSpec((GATHER_W, dw), index_map=lambda i: (i, 0))],
            core_axis_name=("c", "s"),
            dimension_semantics=(pltpu.PARALLEL,),
        )(idx_hbm, out_hbm)

    return gather_kernel(tok_table, x_flat.reshape(1, n_rows))


def _ln_math(g_ref, seg_ref, pos_ref, seg0_ref, segd_ref, gam_ref, bet_ref):
    gi = g_ref[...]
    lo = jax.lax.bitcast_convert_type(
        jax.lax.shift_left(gi, jnp.int32(16)), jnp.float32)
    hi = jax.lax.bitcast_convert_type(
        jnp.bitwise_and(gi, jnp.int32(-65536)), jnp.float32)
    g = jnp.concatenate([lo, hi], axis=-1)
    h = g + pos_ref[...]
    segb = seg_ref[...]
    h = h + seg0_ref[...] + segb * segd_ref[...]
    ones = jnp.ones((D, D), jnp.float32)
    dims = (((2,), (0,)), ((), ()))
    mu = jax.lax.dot_general(h, ones, dims) * (1.0 / D)
    sq = jax.lax.dot_general(h * h, ones, dims) * (1.0 / D)
    var = sq - mu * mu
    return (h - mu) * jax.lax.rsqrt(var + 1e-5) * gam_ref[...] + bet_ref[...]


def _ln_body(prev_ref, g_ref, seg_ref, pos_ref, seg0_ref, segd_ref, gam_ref,
             bet_ref, o_ref, tok_ref):
    del prev_ref
    r = _ln_math(g_ref, seg_ref, pos_ref, seg0_ref, segd_ref, gam_ref, bet_ref)
    o_ref[...] = r
    tok_ref[...] = r[0, :8, :]


def _first_ln_body(g_ref, seg_ref, pos_ref, seg0_ref, segd_ref, gam_ref,
                   bet_ref, o_ref, tok_ref):
    r = _ln_math(g_ref, seg_ref, pos_ref, seg0_ref, segd_ref, gam_ref, bet_ref)
    o_ref[...] = r
    tok_ref[...] = r[0, :8, :]


def _tc_add_ln_chunk(prev, gathered_c, segf_c, pos3, seg0, segd, gamma, beta,
                     chunk):
    base = chunk * (CB // BB)
    small = [
        pl.BlockSpec((1, S, D), lambda i: (0, 0, 0)),
        pl.BlockSpec((1, 1, D), lambda i: (0, 0, 0)),
        pl.BlockSpec((1, 1, D), lambda i: (0, 0, 0)),
        pl.BlockSpec((1, 1, D), lambda i: (0, 0, 0)),
        pl.BlockSpec((1, 1, D), lambda i: (0, 0, 0)),
    ]
    data = [
        pl.BlockSpec((BB, S, D // 2), lambda i: (i, 0, 0)),
        pl.BlockSpec((BB, S, 1), lambda i: (i, 0, 0)),
    ]
    out_shapes = (
        jax.ShapeDtypeStruct((B, S, D), jnp.float32),
        jax.ShapeDtypeStruct((8, D), jnp.float32),
    )
    out_specs = (
        pl.BlockSpec((BB, S, D), lambda i: (base + i, 0, 0)),
        pl.BlockSpec((8, D), lambda i: (0, 0)),
    )
    if prev is None:
        return pl.pallas_call(
            _first_ln_body,
            grid=(CB // BB,),
            in_specs=data + small,
            out_specs=out_specs,
            out_shape=out_shapes,
        )(gathered_c, segf_c, pos3, seg0, segd, gamma, beta)
    return pl.pallas_call(
        _ln_body,
        grid=(CB // BB,),
        in_specs=[pl.BlockSpec((1, 8, D), lambda i: (0, 0, 0))] + data + small,
        out_specs=out_specs,
        out_shape=out_shapes,
        input_output_aliases={0: 0},
    )(prev, gathered_c, segf_c, pos3, seg0, segd, gamma, beta)


def kernel(x, seg, tok_table, pos_table, seg_table, ln_gamma, ln_beta):
    x_flat = x.reshape(-1).astype(jnp.int32)
    tokbf = tok_table.astype(jnp.bfloat16)
    vocab = tok_table.shape[0]
    packed = jnp.stack([tokbf[:, : D // 2], tokbf[:, D // 2:]], axis=-1)
    toki = jax.lax.bitcast_convert_type(packed, jnp.int32)
    toki = toki.reshape(vocab, D // 2)
    segf = seg.astype(jnp.float32).reshape(B, S, 1)
    pos3 = pos_table[:S].reshape(1, S, D)
    seg0 = seg_table[0].reshape(1, 1, D)
    segd = (seg_table[1] - seg_table[0]).reshape(1, 1, D)
    gamma = ln_gamma.reshape(1, 1, D)
    beta = ln_beta.reshape(1, 1, D)

    def xslice(c):
        return jax.lax.dynamic_slice_in_dim(x_flat, c * CB * S, CB * S)

    gathers = [
        _sc_gather(toki, xslice(0), CB * S).reshape(CB, S, D // 2),
        _sc_gather(toki, xslice(1), CB * S).reshape(CB, S, D // 2),
    ]
    out = None
    for c in range(NCH):
        segf_c = jax.lax.dynamic_slice_in_dim(segf, c * CB, CB)
        out, tok = _tc_add_ln_chunk(out, gathers[c], segf_c, pos3, seg0, segd,
                                    gamma, beta, c)
        if c + 2 < NCH:
            xs, _ = jax.lax.optimization_barrier((xslice(c + 2), tok))
            gathers.append(
                _sc_gather(toki, xs, CB * S).reshape(CB, S, D // 2))
    return out


# speedup vs baseline: 1.0346x; 1.0346x over previous
import functools

import jax
import jax.numpy as jnp
from jax.experimental import pallas as pl
from jax.experimental.pallas import tpu as pltpu
from jax.experimental.pallas import tpu_sc as plsc

B = 4096
S = 200
D = 128
TOKS = B * S
GATHER_W = 128
BB = 32


def _sc_gather(tok_table, x_flat, n_rows):
    mesh = plsc.VectorSubcoreMesh(core_axis_name="c", subcore_axis_name="s")
    num_windows = n_rows // GATHER_W
    dw = tok_table.shape[1]

    @functools.partial(
        pl.kernel,
        out_type=jax.ShapeDtypeStruct((n_rows, dw), jnp.int32),
        mesh=mesh,
        compiler_params=pltpu.CompilerParams(use_tc_tiling_on_sc=False),
    )
    def gather_kernel(tok_hbm, idx_hbm, out_hbm):
        def body(idx_vmem, out_vmem):
            pltpu.sync_copy(tok_hbm.at[idx_vmem.at[0]], out_vmem)

        pltpu.emit_pipeline(
            body,
            grid=(num_windows,),
            in_specs=[pl.BlockSpec((1, GATHER_W), index_map=lambda i: (0, i))],
            out_specs=[pl.BlockSpec((GATHER_W, dw), index_map=lambda i: (i, 0))],
            core_axis_name=("c", "s"),
            dimension_semantics=(pltpu.PARALLEL,),
        )(idx_hbm, out_hbm)

    return gather_kernel(tok_table, x_flat.reshape(1, n_rows))


def _ln_body(g_ref, seg_ref, pos_ref, segd_ref, gam_ref, bet_ref, o_ref):
    gi = g_ref[...]
    lo = jax.lax.bitcast_convert_type(
        jax.lax.shift_left(gi, jnp.int32(16)), jnp.float32)
    hi = jax.lax.bitcast_convert_type(
        jnp.bitwise_and(gi, jnp.int32(-65536)), jnp.float32)
    g = jnp.concatenate([lo, hi], axis=-1)
    segb = seg_ref[...]
    h = g + pos_ref[...] + segb * segd_ref[...]
    ones = jnp.full((D, D), 1.0 / D, jnp.float32)
    dims = (((2,), (0,)), ((), ()))
    mu = jax.lax.dot_general(h, ones, dims)
    sq = jax.lax.dot_general(h * h, ones, dims)
    var = sq - mu * mu
    o_ref[...] = (h - mu) * jax.lax.rsqrt(var + 1e-5) * gam_ref[...] + bet_ref[...]


def kernel(x, seg, tok_table, pos_table, seg_table, ln_gamma, ln_beta):
    x_flat = x.reshape(-1).astype(jnp.int32)
    tokbf = tok_table.astype(jnp.bfloat16)
    vocab = tok_table.shape[0]
    packed = jnp.stack([tokbf[:, : D // 2], tokbf[:, D // 2:]], axis=-1)
    toki = jax.lax.bitcast_convert_type(packed, jnp.int32).reshape(vocab, D // 2)
    segf = seg.astype(jnp.float32).reshape(B, S, 1)
    pos3 = (pos_table[:S] + seg_table[0][None, :]).reshape(1, S, D)
    segd = (seg_table[1] - seg_table[0]).reshape(1, 1, D)
    gamma = ln_gamma.reshape(1, 1, D)
    beta = ln_beta.reshape(1, 1, D)

    gathered = _sc_gather(toki, x_flat, TOKS).reshape(B, S, D // 2)
    return pl.pallas_call(
        _ln_body,
        grid=(B // BB,),
        in_specs=[
            pl.BlockSpec((BB, S, D // 2), lambda i: (i, 0, 0)),
            pl.BlockSpec((BB, S, 1), lambda i: (i, 0, 0)),
            pl.BlockSpec((1, S, D), lambda i: (0, 0, 0)),
            pl.BlockSpec((1, 1, D), lambda i: (0, 0, 0)),
            pl.BlockSpec((1, 1, D), lambda i: (0, 0, 0)),
            pl.BlockSpec((1, 1, D), lambda i: (0, 0, 0)),
        ],
        out_specs=pl.BlockSpec((BB, S, D), lambda i: (i, 0, 0)),
        out_shape=jax.ShapeDtypeStruct((B, S, D), jnp.float32),
    )(gathered, segf, pos3, segd, gamma, beta)


# speedup vs baseline: 1.4522x; 1.4037x over previous
import functools

import jax
import jax.numpy as jnp
from jax.experimental import pallas as pl
from jax.experimental.pallas import tpu as pltpu
from jax.experimental.pallas import tpu_sc as plsc

B = 4096
S = 200
D = 128
TOKS = B * S
GATHER_W = 128
BB = 32


def _sc_gather(tok_table, x_flat, n_rows):
    mesh = plsc.VectorSubcoreMesh(core_axis_name="c", subcore_axis_name="s")
    num_windows = n_rows // GATHER_W

    @functools.partial(
        pl.kernel,
        out_type=jax.ShapeDtypeStruct((n_rows, D), jnp.float32),
        mesh=mesh,
    )
    def gather_kernel(tok_hbm, idx_hbm, out_hbm):
        def body(idx_vmem, out_vmem):
            pltpu.sync_copy(tok_hbm.at[idx_vmem.at[0]], out_vmem)

        pltpu.emit_pipeline(
            body,
            grid=(num_windows,),
            in_specs=[pl.BlockSpec((1, GATHER_W), index_map=lambda i: (0, i))],
            out_specs=[pl.BlockSpec((GATHER_W, D), index_map=lambda i: (i, 0))],
            core_axis_name=("c", "s"),
            dimension_semantics=(pltpu.PARALLEL,),
        )(idx_hbm, out_hbm)

    return gather_kernel(tok_table, x_flat.reshape(1, n_rows))


def _ln_body(g_ref, seg_ref, pos_ref, segd_ref, gam_ref, bet_ref, o_ref):
    segb = seg_ref[...]
    h = g_ref[...] + pos_ref[...] + segb * segd_ref[...]
    ones = jnp.full((D, D), 1.0 / D, jnp.float32)
    dims = (((2,), (0,)), ((), ()))
    mu = jax.lax.dot_general(h, ones, dims)
    sq = jax.lax.dot_general(h * h, ones, dims)
    var = sq - mu * mu
    o_ref[...] = (h - mu) * jax.lax.rsqrt(var + 1e-5) * gam_ref[...] + bet_ref[...]


def kernel(x, seg, tok_table, pos_table, seg_table, ln_gamma, ln_beta):
    x_flat = x.reshape(-1).astype(jnp.int32)
    segf = seg.astype(jnp.float32).reshape(B, S, 1)
    pos3 = (pos_table[:S] + seg_table[0][None, :]).reshape(1, S, D)
    segd = (seg_table[1] - seg_table[0]).reshape(1, 1, D)
    gamma = ln_gamma.reshape(1, 1, D)
    beta = ln_beta.reshape(1, 1, D)

    gathered = _sc_gather(tok_table, x_flat, TOKS).reshape(B, S, D)
    return pl.pallas_call(
        _ln_body,
        grid=(B // BB,),
        in_specs=[
            pl.BlockSpec((BB, S, D), lambda i: (i, 0, 0)),
            pl.BlockSpec((BB, S, 1), lambda i: (i, 0, 0)),
            pl.BlockSpec((1, S, D), lambda i: (0, 0, 0)),
            pl.BlockSpec((1, 1, D), lambda i: (0, 0, 0)),
            pl.BlockSpec((1, 1, D), lambda i: (0, 0, 0)),
            pl.BlockSpec((1, 1, D), lambda i: (0, 0, 0)),
        ],
        out_specs=pl.BlockSpec((BB, S, D), lambda i: (i, 0, 0)),
        out_shape=jax.ShapeDtypeStruct((B, S, D), jnp.float32),
    )(gathered, segf, pos3, segd, gamma, beta)
